# baseline (speedup 1.0000x reference)
import functools

import jax
import jax.numpy as jnp
from jax import lax
from jax.experimental import pallas as pl
from jax.experimental.pallas import tpu as pltpu
from jax.experimental.pallas import tpu_sc as plsc

N_NODES = 10000
N_EDGES = 320000
D = 128

NC = 2
NS = 16
NW = NC * NS
E_PER_TILE = N_EDGES // NW
CHUNK = 80
N_CHUNKS = E_PER_TILE // CHUNK
NA = (N_CHUNKS + 1) // 2
NB = N_CHUNKS // 2
ACC_ROWS = 10240
ROWS_PER_TILE = ACC_ROWS // NS
ZROWS = 32
WROWS = 128


def _mm_combine_body(p_ref, q_ref, w_ref, b_ref, o_ref):
    o_ref[...] = jnp.dot(p_ref[0] + q_ref[0], w_ref[...],
                         preferred_element_type=jnp.float32) + b_ref[...]


def _sc_body(support_hbm, dst_hbm, src_hbm, out_hbm,
             sA0, sA1, sA2, sA3, dA0, dA1, dA2, dA3,
             sB0, sB1, sB2, sB3, dB0, dB1, dB2, dB3,
             bufA0, bufA1, bufB0, bufB1, zbuf_v, acc_sh,
             semGA, semSA0, semSA1, semIA0, semIA1, semIA2, semIA3,
             semGB, semSB0, semSB1, semIB0, semIB1, semIB2, semIB3):
    c = lax.axis_index("c")
    s = lax.axis_index("s")
    wid = c * NS + s

    ebase = wid * E_PER_TILE

    def make_slot(offs, srcP, dstP, bufs, semG, semS, semI):
        def islice(k):
            return pl.ds(ebase + (2 * k + offs) * CHUNK, CHUNK)

        def pre_idx(k, q):
            pltpu.async_copy(src_hbm.at[islice(k)], srcP[q], semI[q])
            pltpu.async_copy(dst_hbm.at[islice(k)], dstP[q], semI[q])

        def wait_idx(k, q):
            pltpu.make_async_copy(src_hbm.at[islice(k)], srcP[q],
                                  semI[q]).wait()
            pltpu.make_async_copy(dst_hbm.at[islice(k)], dstP[q],
                                  semI[q]).wait()

        def gather(p, q):
            pltpu.async_copy(support_hbm.at[srcP[q]], bufs[p], semG)

        def wait_g(p, q):
            pltpu.make_async_copy(support_hbm.at[srcP[q]], bufs[p],
                                  semG).wait()

        def scat(p, q):
            pltpu.async_copy(bufs[p], acc_sh.at[dstP[q]], semS[p], add=True)

        def wait_s(p, q):
            pltpu.make_async_copy(bufs[p], acc_sh.at[dstP[q]], semS[p]).wait()

        def prologue0():
            for q in range(4):
                pre_idx(q, q)
            wait_idx(0, 0)
            gather(0, 0)

        def prologue1():
            wait_g(0, 0)
            scat(0, 0)
            wait_idx(1, 1)
            gather(1, 1)
            wait_g(1, 1)
            scat(1, 1)
            wait_s(0, 0)
            pre_idx(4, 0)
            wait_idx(2, 2)
            gather(0, 2)

        def step(k, p, q):
            wait_g(p, q)
            scat(p, q)
            wait_s(1 - p, (q + 3) % 4)
            pre_idx(k + 3, (q + 3) % 4)
            wait_idx(k + 1, (q + 1) % 4)
            gather(1 - p, (q + 1) % 4)

        def step_nopre(k, p, q):
            wait_g(p, q)
            scat(p, q)
            wait_s(1 - p, (q + 3) % 4)
            wait_idx(k + 1, (q + 1) % 4)
            gather(1 - p, (q + 1) % 4)

        def fin(p, q):
            wait_g(p, q)
            scat(p, q)
            wait_s(1 - p, (q + 3) % 4)
            wait_s(p, q)

        return prologue0, prologue1, step, step_nopre, fin

    pro0A, pro1A, stepA, stepnA, finA = make_slot(
        0, (sA0, sA1, sA2, sA3), (dA0, dA1, dA2, dA3),
        (bufA0, bufA1), semGA, (semSA0, semSA1),
        (semIA0, semIA1, semIA2, semIA3))
    pro0B, pro1B, stepB, stepnB, finB = make_slot(
        1, (sB0, sB1, sB2, sB3), (dB0, dB1, dB2, dB3),
        (bufB0, bufB1), semGB, (semSB0, semSB1),
        (semIB0, semIB1, semIB2, semIB3))

    pro0A()
    pro0B()

    zero16 = jnp.zeros((16,), jnp.float32)

    def zstore(i, carry):
        zbuf_v[i // 8, pl.ds((i % 8) * 16, 16)] = zero16
        return carry

    lax.fori_loop(0, ZROWS * (D // 16), zstore, 0)
    for t in range(ROWS_PER_TILE // ZROWS):
        pltpu.sync_copy(zbuf_v,
                        acc_sh.at[pl.ds(s * ROWS_PER_TILE + t * ZROWS, ZROWS)])
    plsc.subcore_barrier()

    pro1A()
    pro1B()

    def pipe_body(t, carry):
        for u in range(4):
            k = 4 * t + 2 + u
            p, q = (2 + u) % 2, (2 + u) % 4
            stepA(k, p, q)
            stepB(k, p, q)
        return carry

    lax.fori_loop(0, 14, pipe_body, 0)
    stepA(58, 0, 2)
    stepB(58, 0, 2)
    stepA(59, 1, 3)
    stepnA(60, 0, 0)
    stepnA(61, 1, 1)
    finA(0, 2)
    stepnB(59, 1, 3)
    stepnB(60, 0, 0)
    finB(1, 1)
    plsc.subcore_barrier()

    cps = []
    for t in range(ROWS_PER_TILE // WROWS):
        r = s * ROWS_PER_TILE + t * WROWS
        cps.append(pltpu.async_copy(acc_sh.at[pl.ds(r, WROWS)],
                                    out_hbm.at[pl.ds(c * ACC_ROWS + r, WROWS)],
                                    semGA))
    for cp in cps:
        cp.wait()


_sc_aggregate = functools.partial(
    pl.kernel,
    out_type=jax.ShapeDtypeStruct((NC * ACC_ROWS, D), jnp.float32),
    mesh=plsc.VectorSubcoreMesh(core_axis_name="c", subcore_axis_name="s"),
    scratch_types=(
        [pltpu.VMEM((CHUNK,), jnp.int32)] * 16
        + [pltpu.VMEM((CHUNK, D), jnp.float32)] * 4
        + [
            pltpu.VMEM((ZROWS, D), jnp.float32),
            pltpu.VMEM_SHARED((ACC_ROWS, D), jnp.float32),
        ]
        + [pltpu.SemaphoreType.DMA] * 14
    ),
)(_sc_body)


@jax.jit
def kernel(input, edge_index, W, b):
    ei = edge_index.astype(jnp.int32)
    dst = ei[0]
    src = ei[1]

    partial = _sc_aggregate(input, dst, src)

    partial3 = partial.reshape(NC, ACC_ROWS, D)
    out = pl.pallas_call(
        _mm_combine_body,
        grid=(10,),
        in_specs=[
            pl.BlockSpec((1, 1000, D), lambda i: (0, i, 0)),
            pl.BlockSpec((1, 1000, D), lambda i: (1, i, 0)),
            pl.BlockSpec((D, D), lambda i: (0, 0)),
            pl.BlockSpec((1, D), lambda i: (0, 0)),
        ],
        out_specs=pl.BlockSpec((1000, D), lambda i: (i, 0)),
        out_shape=jax.ShapeDtypeStruct((N_NODES, D), jnp.float32),
    )(partial3, partial3, W, b.reshape(1, D))
    return out


# speedup vs baseline: 1.0511x; 1.0511x over previous
import functools

import jax
import jax.numpy as jnp
from jax import lax
from jax.experimental import pallas as pl
from jax.experimental.pallas import tpu as pltpu
from jax.experimental.pallas import tpu_sc as plsc

N_NODES = 10000
N_EDGES = 320000
D = 128

NC = 2
NS = 16
NW = NC * NS
E_PER_TILE = N_EDGES // NW
CHUNK = 128
N_FULL = E_PER_TILE // CHUNK
TAIL = E_PER_TILE - N_FULL * CHUNK
ACC_ROWS = 10240
ROWS_PER_TILE = ACC_ROWS // NS
ZROWS = 64
WROWS = 128


def _mm_combine_body(p_ref, q_ref, w_ref, b_ref, o_ref):
    o_ref[...] = jnp.dot(p_ref[0] + q_ref[0], w_ref[...],
                         preferred_element_type=jnp.float32) + b_ref[...]


def _sc_body(support_hbm, dst_hbm, src_hbm, out_hbm,
             srcA0, srcA1, dstA0, dstA1, srcB0, srcB1, dstB0, dstB1,
             srcT, dstT, bufA, bufB, bufT, zbuf_v,
             acc_sh, semA, semB, semIA, semIB):
    c = lax.axis_index("c")
    s = lax.axis_index("s")
    wid = c * NS + s

    ebase = wid * E_PER_TILE

    def make_slot(offs, srcP, dstP, buf, semG, semI):
        def islice(k):
            return pl.ds(ebase + (2 * k + offs) * CHUNK, CHUNK)

        def pre_idx(k, p):
            pltpu.async_copy(src_hbm.at[islice(k)], srcP[p], semI)
            pltpu.async_copy(dst_hbm.at[islice(k)], dstP[p], semI)

        def wait_idx(k, p):
            pltpu.make_async_copy(src_hbm.at[islice(k)], srcP[p], semI).wait()
            pltpu.make_async_copy(dst_hbm.at[islice(k)], dstP[p], semI).wait()

        def gather(p):
            pltpu.async_copy(support_hbm.at[srcP[p]], buf, semG)

        def wait_g(p):
            pltpu.make_async_copy(support_hbm.at[srcP[p]], buf, semG).wait()

        def scatter(p):
            pltpu.sync_copy(buf, acc_sh.at[dstP[p]], add=True)

        def prologue():
            pre_idx(0, 0)
            pre_idx(1, 1)
            wait_idx(0, 0)
            gather(0)

        def step(k, p):
            wait_g(p)
            scatter(p)
            wait_idx(k + 1, 1 - p)
            gather(1 - p)
            pre_idx(k + 2, p)

        def fin():
            wait_g(1)
            scatter(1)
            wait_idx(38, 0)
            gather(0)
            wait_g(0)
            scatter(0)

        return prologue, step, fin

    proA, stepA, finA = make_slot(0, (srcA0, srcA1), (dstA0, dstA1),
                                  bufA, semA, semIA)
    proB, stepB, finB = make_slot(1, (srcB0, srcB1), (dstB0, dstB1),
                                  bufB, semB, semIB)

    proA()
    proB()

    zero16 = jnp.zeros((16,), jnp.float32)

    def zstore(i, carry):
        zbuf_v[i // 8, pl.ds((i % 8) * 16, 16)] = zero16
        return carry

    lax.fori_loop(0, ZROWS * (D // 16), zstore, 0)
    for t in range(ROWS_PER_TILE // ZROWS):
        pltpu.sync_copy(zbuf_v,
                        acc_sh.at[pl.ds(s * ROWS_PER_TILE + t * ZROWS, ZROWS)])
    plsc.subcore_barrier()

    def pipe_body(r, carry):
        k = 2 * r
        stepA(k, 0)
        stepB(k, 0)
        stepA(k + 1, 1)
        stepB(k + 1, 1)
        return carry

    lax.fori_loop(0, 18, pipe_body, 0)
    stepA(36, 0)
    stepB(36, 0)
    finA()
    finB()

    pltpu.sync_copy(src_hbm.at[pl.ds(ebase + N_FULL * CHUNK, TAIL)], srcT)
    pltpu.sync_copy(dst_hbm.at[pl.ds(ebase + N_FULL * CHUNK, TAIL)], dstT)
    pltpu.async_copy(support_hbm.at[srcT], bufT, semA).wait()
    pltpu.sync_copy(bufT, acc_sh.at[dstT], add=True)
    plsc.subcore_barrier()

    cps = []
    for t in range(ROWS_PER_TILE // WROWS):
        r = s * ROWS_PER_TILE + t * WROWS
        cps.append(pltpu.async_copy(acc_sh.at[pl.ds(r, WROWS)],
                                    out_hbm.at[pl.ds(c * ACC_ROWS + r, WROWS)],
                                    semA))
    for cp in cps:
        cp.wait()


_sc_aggregate = functools.partial(
    pl.kernel,
    out_type=jax.ShapeDtypeStruct((NC * ACC_ROWS, D), jnp.float32),
    mesh=plsc.VectorSubcoreMesh(core_axis_name="c", subcore_axis_name="s"),
    scratch_types=[
        pltpu.VMEM((CHUNK,), jnp.int32),
        pltpu.VMEM((CHUNK,), jnp.int32),
        pltpu.VMEM((CHUNK,), jnp.int32),
        pltpu.VMEM((CHUNK,), jnp.int32),
        pltpu.VMEM((CHUNK,), jnp.int32),
        pltpu.VMEM((CHUNK,), jnp.int32),
        pltpu.VMEM((CHUNK,), jnp.int32),
        pltpu.VMEM((CHUNK,), jnp.int32),
        pltpu.VMEM((TAIL,), jnp.int32),
        pltpu.VMEM((TAIL,), jnp.int32),
        pltpu.VMEM((CHUNK, D), jnp.float32),
        pltpu.VMEM((CHUNK, D), jnp.float32),
        pltpu.VMEM((TAIL, D), jnp.float32),
        pltpu.VMEM((ZROWS, D), jnp.float32),
        pltpu.VMEM_SHARED((ACC_ROWS, D), jnp.float32),
        pltpu.SemaphoreType.DMA,
        pltpu.SemaphoreType.DMA,
        pltpu.SemaphoreType.DMA,
        pltpu.SemaphoreType.DMA,
    ],
)(_sc_body)


@jax.jit
def kernel(input, edge_index, W, b):
    ei = edge_index.astype(jnp.int32)
    dst = ei[0]
    src = ei[1]

    partial = _sc_aggregate(input, dst, src)

    partial3 = partial.reshape(NC, ACC_ROWS, D)
    out = pl.pallas_call(
        _mm_combine_body,
        grid=(10,),
        in_specs=[
            pl.BlockSpec((1, 1000, D), lambda i: (0, i, 0)),
            pl.BlockSpec((1, 1000, D), lambda i: (1, i, 0)),
            pl.BlockSpec((D, D), lambda i: (0, 0)),
            pl.BlockSpec((1, D), lambda i: (0, 0)),
        ],
        out_specs=pl.BlockSpec((1000, D), lambda i: (i, 0)),
        out_shape=jax.ShapeDtypeStruct((N_NODES, D), jnp.float32),
    )(partial3, partial3, W, b.reshape(1, D))
    return out
